# baseline (speedup 1.0000x reference)
import jax
import jax.numpy as jnp
from jax.experimental import pallas as pl
from jax.experimental.pallas import tpu as pltpu

G, V = 8, 512
GV = G * V
D = 128
TT = 256


def _gumbel_z(l, temp):
    gum = -jnp.log(-jnp.log(l + 1e-05) + 1e-05)
    return (l + gum) / temp


def _vq_kernel(temp_ref, x_ref, w_ref, b_ref, cb_ref, logits_ref, out_ref):
    x = x_ref[0]
    temp = temp_ref[0]
    logits = jax.lax.dot_general(
        w_ref[...], x, (((1,), (1,)), ((), ())),
        preferred_element_type=jnp.float32)
    logits = logits + b_ref[...]
    logits_ref[0] = logits
    rows = jax.lax.broadcasted_iota(jnp.int32, (V, TT), 0)
    for g in range(G):
        lg = logits[g * V:(g + 1) * V, :]
        m1 = jnp.max(lg, axis=0)
        eq1 = lg == m1[None, :]
        m2 = jnp.max(jnp.where(eq1, -jnp.inf, lg), axis=0)
        collapse = jnp.exp(_gumbel_z(m2, temp) - _gumbel_z(m1, temp)) >= 1.0
        winner = eq1 | (collapse[None, :] & (lg == m2[None, :]))
        idx = jnp.min(jnp.where(winner, rows, V), axis=0)
        onehot = (jax.lax.broadcasted_iota(jnp.int32, (TT, V), 1)
                  == idx[:, None]).astype(jnp.float32)
        cb_g = cb_ref[g * D:(g + 1) * D, :]
        hard = jax.lax.dot_general(
            onehot, cb_g, (((1,), (1,)), ((), ())),
            preferred_element_type=jnp.float32)
        out_ref[0, :, g * D:(g + 1) * D] = hard


def kernel(inputs, W, b, codebooks, temp):
    bsize, timesteps, channels = inputs.shape
    b2 = b.reshape(GV, 1)
    cb = codebooks.reshape(G * D, V)
    temp1 = jnp.asarray(temp, jnp.float32).reshape(1)
    logits_flat, out = pl.pallas_call(
        _vq_kernel,
        grid=(bsize, timesteps // TT),
        in_specs=[
            pl.BlockSpec(memory_space=pltpu.SMEM),
            pl.BlockSpec((1, TT, channels), lambda i, j: (i, j, 0)),
            pl.BlockSpec((GV, channels), lambda i, j: (0, 0)),
            pl.BlockSpec((GV, 1), lambda i, j: (0, 0)),
            pl.BlockSpec((G * D, V), lambda i, j: (0, 0)),
        ],
        out_specs=[
            pl.BlockSpec((1, GV, TT), lambda i, j: (i, 0, j)),
            pl.BlockSpec((1, TT, channels), lambda i, j: (i, j, 0)),
        ],
        out_shape=[
            jax.ShapeDtypeStruct((bsize, GV, timesteps), jnp.float32),
            jax.ShapeDtypeStruct((bsize, timesteps, channels), jnp.float32),
        ],
    )(temp1, inputs, W, b2, cb)
    logits = logits_flat.reshape(bsize, G, V, timesteps)
    return out, logits


# speedup vs baseline: 3.6837x; 3.6837x over previous
import jax
import jax.numpy as jnp
from jax.experimental import pallas as pl
from jax.experimental.pallas import tpu as pltpu

G, V = 8, 512
GV = G * V
D = 128
TT = 256


def _gumbel_z(l, temp):
    gum = -jnp.log(-jnp.log(l + 1e-05) + 1e-05)
    return (l + gum) / temp


def _vq_kernel(temp_ref, x_ref, w_ref, b_ref, cb_ref, logits_ref, out_ref):
    x = x_ref[0]
    temp = temp_ref[0]
    logits = jax.lax.dot_general(
        w_ref[...], x, (((1,), (1,)), ((), ())),
        preferred_element_type=jnp.float32)
    logits = logits + b_ref[...]
    logits_ref[0] = logits
    for g in range(G):
        lg = logits[g * V:(g + 1) * V, :]
        j1 = jnp.argmax(lg, axis=0)
        m1 = jnp.max(lg, axis=0)
        masked = jnp.where(lg == m1[None, :], -jnp.inf, lg)
        j2 = jnp.argmax(masked, axis=0)
        m2 = jnp.max(masked, axis=0)
        collapse = jnp.exp(_gumbel_z(m2, temp) - _gumbel_z(m1, temp)) >= 1.0
        idx = jnp.where(collapse, jnp.minimum(j1, j2), j1)
        onehot = (jax.lax.broadcasted_iota(jnp.int32, (TT, V), 1)
                  == idx[:, None]).astype(jnp.float32)
        cb_g = cb_ref[g * D:(g + 1) * D, :]
        hard = jax.lax.dot_general(
            onehot, cb_g, (((1,), (1,)), ((), ())),
            preferred_element_type=jnp.float32)
        out_ref[0, :, g * D:(g + 1) * D] = hard


def kernel(inputs, W, b, codebooks, temp):
    bsize, timesteps, channels = inputs.shape
    b2 = b.reshape(GV, 1)
    cb = codebooks.reshape(G * D, V)
    temp1 = jnp.asarray(temp, jnp.float32).reshape(1)
    logits_flat, out = pl.pallas_call(
        _vq_kernel,
        grid=(bsize, timesteps // TT),
        in_specs=[
            pl.BlockSpec(memory_space=pltpu.SMEM),
            pl.BlockSpec((1, TT, channels), lambda i, j: (i, j, 0)),
            pl.BlockSpec((GV, channels), lambda i, j: (0, 0)),
            pl.BlockSpec((GV, 1), lambda i, j: (0, 0)),
            pl.BlockSpec((G * D, V), lambda i, j: (0, 0)),
        ],
        out_specs=[
            pl.BlockSpec((1, GV, TT), lambda i, j: (i, 0, j)),
            pl.BlockSpec((1, TT, channels), lambda i, j: (i, j, 0)),
        ],
        out_shape=[
            jax.ShapeDtypeStruct((bsize, GV, timesteps), jnp.float32),
            jax.ShapeDtypeStruct((bsize, timesteps, channels), jnp.float32),
        ],
    )(temp1, inputs, W, b2, cb)
    logits = logits_flat.reshape(bsize, G, V, timesteps)
    return out, logits


# speedup vs baseline: 3.7734x; 1.0244x over previous
import jax
import jax.numpy as jnp
from jax.experimental import pallas as pl
from jax.experimental.pallas import tpu as pltpu

G, V = 8, 512
GV = G * V
D = 128
TT = 512


def _gumbel_z(l, temp):
    gum = -jnp.log(-jnp.log(l + 1e-05) + 1e-05)
    return (l + gum) / temp


def _vq_kernel(temp_ref, x_ref, w_ref, b_ref, cb_ref, logits_ref, out_ref):
    x = x_ref[0]
    temp = temp_ref[0]
    logits = jax.lax.dot_general(
        w_ref[...], x, (((1,), (1,)), ((), ())),
        preferred_element_type=jnp.float32)
    logits = logits + b_ref[...]
    logits_ref[0] = logits
    for g in range(G):
        lg = logits[g * V:(g + 1) * V, :]
        j1 = jnp.argmax(lg, axis=0)
        m1 = jnp.max(lg, axis=0)
        masked = jnp.where(lg == m1[None, :], -jnp.inf, lg)
        j2 = jnp.argmax(masked, axis=0)
        m2 = jnp.max(masked, axis=0)
        collapse = jnp.exp(_gumbel_z(m2, temp) - _gumbel_z(m1, temp)) >= 1.0
        idx = jnp.where(collapse, jnp.minimum(j1, j2), j1)
        onehot = (jax.lax.broadcasted_iota(jnp.int32, (TT, V), 1)
                  == idx[:, None]).astype(jnp.float32)
        cb_g = cb_ref[g * D:(g + 1) * D, :]
        hard = jax.lax.dot_general(
            onehot, cb_g, (((1,), (1,)), ((), ())),
            preferred_element_type=jnp.float32)
        out_ref[0, :, g * D:(g + 1) * D] = hard


def kernel(inputs, W, b, codebooks, temp):
    bsize, timesteps, channels = inputs.shape
    b2 = b.reshape(GV, 1)
    cb = codebooks.reshape(G * D, V)
    temp1 = jnp.asarray(temp, jnp.float32).reshape(1)
    logits_flat, out = pl.pallas_call(
        _vq_kernel,
        grid=(bsize, timesteps // TT),
        in_specs=[
            pl.BlockSpec(memory_space=pltpu.SMEM),
            pl.BlockSpec((1, TT, channels), lambda i, j: (i, j, 0)),
            pl.BlockSpec((GV, channels), lambda i, j: (0, 0)),
            pl.BlockSpec((GV, 1), lambda i, j: (0, 0)),
            pl.BlockSpec((G * D, V), lambda i, j: (0, 0)),
        ],
        out_specs=[
            pl.BlockSpec((1, GV, TT), lambda i, j: (i, 0, j)),
            pl.BlockSpec((1, TT, channels), lambda i, j: (i, j, 0)),
        ],
        out_shape=[
            jax.ShapeDtypeStruct((bsize, GV, timesteps), jnp.float32),
            jax.ShapeDtypeStruct((bsize, timesteps, channels), jnp.float32),
        ],
        compiler_params=pltpu.CompilerParams(
            vmem_limit_bytes=64 * 1024 * 1024),
    )(temp1, inputs, W, b2, cb)
    logits = logits_flat.reshape(bsize, G, V, timesteps)
    return out, logits
